# baseline (speedup 1.0000x reference)
import jax
import jax.numpy as jnp
from jax import lax
from jax.experimental import pallas as pl
from jax.experimental.pallas import tpu as pltpu
from jax.experimental.pallas import tpu_sc as plsc

B = 16384
D = 64
NC = 2
NS = 16
NW = NC * NS
BPW = B // NW
L = 16


def _body(user_hbm, item_hbm, utab_hbm, itab_hbm, out_hbm,
          uidx_v, iidx_v, ubuf_v, ibuf_v, out_v, sem_u, sem_i):
    wid = lax.axis_index("s") * NC + lax.axis_index("c")
    base = wid * BPW

    pltpu.sync_copy(user_hbm.at[pl.ds(base, BPW)], uidx_v)
    pltpu.sync_copy(item_hbm.at[pl.ds(base, BPW)], iidx_v)

    def zero(g, carry):
        out_v[pl.ds(g * L, L)] = jnp.zeros((L,), jnp.float32)
        return carry
    lax.fori_loop(0, BPW // L, zero, 0)

    for d in range(D):
        cu = pltpu.async_copy(utab_hbm.at[d].at[uidx_v], ubuf_v, sem_u)
        ci = pltpu.async_copy(itab_hbm.at[d].at[iidx_v], ibuf_v, sem_i)
        cu.wait()
        ci.wait()

        def accum(g, carry):
            s = pl.ds(g * L, L)
            out_v[s] = out_v[s] + ubuf_v[s] * ibuf_v[s]
            return carry
        lax.fori_loop(0, BPW // L, accum, 0)

    pltpu.sync_copy(out_v, out_hbm.at[pl.ds(base, BPW)])


@jax.jit
def kernel(user, item, user_table, item_table):
    mesh = plsc.VectorSubcoreMesh(core_axis_name="c", subcore_axis_name="s")
    run = pl.kernel(
        _body,
        out_type=jax.ShapeDtypeStruct((B,), jnp.float32),
        mesh=mesh,
        compiler_params=pltpu.CompilerParams(
            needs_layout_passes=False,
            use_tc_tiling_on_sc=False,
        ),
        scratch_types=[
            pltpu.VMEM((BPW,), jnp.int32),
            pltpu.VMEM((BPW,), jnp.int32),
            pltpu.VMEM((BPW,), jnp.float32),
            pltpu.VMEM((BPW,), jnp.float32),
            pltpu.VMEM((BPW,), jnp.float32),
            pltpu.SemaphoreType.DMA,
            pltpu.SemaphoreType.DMA,
        ],
    )
    return run(user.astype(jnp.int32), item.astype(jnp.int32),
               user_table.T, item_table.T)


# speedup vs baseline: 20.1199x; 20.1199x over previous
import functools

import jax
import jax.numpy as jnp
from jax import lax
from jax.experimental import pallas as pl
from jax.experimental.pallas import tpu as pltpu
from jax.experimental.pallas import tpu_sc as plsc

B = 16384
D = 64
NC = 2
NS = 16
NW = NC * NS
BPW = B // NW
L = 16


def _body(user_hbm, item_hbm, utab_hbm, itab_hbm, out_hbm,
          uidx_v, iidx_v, ub0, ub1, ib0, ib1, out_v,
          sem_u0, sem_u1, sem_i0, sem_i1):
    wid = lax.axis_index("s") * NC + lax.axis_index("c")
    base = wid * BPW

    pltpu.sync_copy(user_hbm.at[pl.ds(base, BPW)], uidx_v)
    pltpu.sync_copy(item_hbm.at[pl.ds(base, BPW)], iidx_v)

    ubufs = (ub0, ub1)
    ibufs = (ib0, ib1)
    usems = (sem_u0, sem_u1)
    isems = (sem_i0, sem_i1)
    lanes = lax.iota(jnp.int32, L)

    def issue(uj, ij, s):
        ustart = pl.multiple_of((uj >> 7) * 128, 128)
        istart = pl.multiple_of((ij >> 7) * 128, 128)
        cu = pltpu.async_copy(utab_hbm.at[:, pl.ds(ustart, 128)],
                              ubufs[s], usems[s])
        ci = pltpu.async_copy(itab_hbm.at[:, pl.ds(istart, 128)],
                              ibufs[s], isems[s])
        return (cu, ci)

    def group(g, carry):
        uvec = uidx_v[pl.ds(g * L, L)]
        ivec = iidx_v[pl.ds(g * L, L)]
        descs = [None, None]
        descs[0] = issue(uvec[0], ivec[0], 0)
        accv = jnp.zeros((L,), jnp.float32)
        for j in range(L):
            s = j & 1
            if j + 1 < L:
                descs[1 - s] = issue(uvec[j + 1], ivec[j + 1], 1 - s)
            cu, ci = descs[s]
            cu.wait()
            ci.wait()
            ucol = jnp.full((L,), uvec[j] & 127, jnp.int32)
            icol = jnp.full((L,), ivec[j] & 127, jnp.int32)
            prod = jnp.zeros((L,), jnp.float32)
            for k in range(D // L):
                rows = k * L + lanes
                gu = plsc.load_gather(ubufs[s], [rows, ucol])
                gi = plsc.load_gather(ibufs[s], [rows, icol])
                prod = prod + gu * gi
            dot = jnp.sum(prod)
            accv = jnp.where(lanes == j, dot, accv)
        out_v[pl.ds(g * L, L)] = accv
        return carry

    lax.fori_loop(0, BPW // L, group, 0)
    pltpu.sync_copy(out_v, out_hbm.at[pl.ds(base, BPW)])


@jax.jit
def kernel(user, item, user_table, item_table):
    mesh = plsc.VectorSubcoreMesh(core_axis_name="c", subcore_axis_name="s")
    run = pl.kernel(
        _body,
        out_type=jax.ShapeDtypeStruct((B,), jnp.float32),
        mesh=mesh,
        compiler_params=pltpu.CompilerParams(
            needs_layout_passes=False,
        ),
        scratch_types=[
            pltpu.VMEM((BPW,), jnp.int32),
            pltpu.VMEM((BPW,), jnp.int32),
            pltpu.VMEM((D, 128), jnp.float32),
            pltpu.VMEM((D, 128), jnp.float32),
            pltpu.VMEM((D, 128), jnp.float32),
            pltpu.VMEM((D, 128), jnp.float32),
            pltpu.VMEM((BPW,), jnp.float32),
            pltpu.SemaphoreType.DMA,
            pltpu.SemaphoreType.DMA,
            pltpu.SemaphoreType.DMA,
            pltpu.SemaphoreType.DMA,
        ],
    )
    return run(user.astype(jnp.int32), item.astype(jnp.int32),
               user_table.T, item_table.T)


# speedup vs baseline: 24.4108x; 1.2133x over previous
import functools

import jax
import jax.numpy as jnp
from jax import lax
from jax.experimental import pallas as pl
from jax.experimental.pallas import tpu as pltpu
from jax.experimental.pallas import tpu_sc as plsc

B = 16384
D = 64
NC = 2
NS = 16
NW = NC * NS
BPW = B // NW
L = 16
NSLOT = 4
AHEAD = 3


def _body(user_hbm, item_hbm, utab_hbm, itab_hbm, out_hbm,
          uidx_v, iidx_v, ub0, ub1, ub2, ub3, ib0, ib1, ib2, ib3, out_v,
          su0, su1, su2, su3, si0, si1, si2, si3):
    wid = lax.axis_index("s") * NC + lax.axis_index("c")
    base = wid * BPW

    pltpu.sync_copy(user_hbm.at[pl.ds(base, BPW)], uidx_v)
    pltpu.sync_copy(item_hbm.at[pl.ds(base, BPW)], iidx_v)

    ubufs = (ub0, ub1, ub2, ub3)
    ibufs = (ib0, ib1, ib2, ib3)
    usems = (su0, su1, su2, su3)
    isems = (si0, si1, si2, si3)
    lanes = lax.iota(jnp.int32, L)

    def issue(uj, ij, s):
        ustart = pl.multiple_of((uj >> 7) * 128, 128)
        istart = pl.multiple_of((ij >> 7) * 128, 128)
        pltpu.async_copy(utab_hbm.at[:, pl.ds(ustart, 128)],
                         ubufs[s], usems[s])
        pltpu.async_copy(itab_hbm.at[:, pl.ds(istart, 128)],
                         ibufs[s], isems[s])

    def wait(s):
        pltpu.make_async_copy(utab_hbm.at[:, pl.ds(0, 128)],
                              ubufs[s], usems[s]).wait()
        pltpu.make_async_copy(itab_hbm.at[:, pl.ds(0, 128)],
                              ibufs[s], isems[s]).wait()

    uvec0 = uidx_v[pl.ds(0, L)]
    ivec0 = iidx_v[pl.ds(0, L)]
    for j in range(AHEAD):
        issue(uvec0[j], ivec0[j], j % NSLOT)

    def group(g, carry):
        uvec = uidx_v[pl.ds(g * L, L)]
        ivec = iidx_v[pl.ds(g * L, L)]
        gn = jnp.minimum(g + 1, BPW // L - 1)
        uvecn = uidx_v[pl.ds(gn * L, L)]
        ivecn = iidx_v[pl.ds(gn * L, L)]
        accv = jnp.zeros((L,), jnp.float32)
        for j in range(L):
            s = j % NSLOT
            wait(s)
            ja = j + AHEAD
            if ja < L:
                una, ina = uvec[ja], ivec[ja]
            else:
                una, ina = uvecn[ja - L], ivecn[ja - L]

            @pl.when(g * L + ja < BPW)
            def _():
                issue(una, ina, ja % NSLOT)

            ucol = jnp.full((L,), uvec[j] & 127, jnp.int32)
            icol = jnp.full((L,), ivec[j] & 127, jnp.int32)
            prod = jnp.zeros((L,), jnp.float32)
            for k in range(D // L):
                rows = k * L + lanes
                gu = plsc.load_gather(ubufs[s], [rows, ucol])
                gi = plsc.load_gather(ibufs[s], [rows, icol])
                prod = prod + gu * gi
            dot = jnp.sum(prod)
            accv = jnp.where(lanes == j, dot, accv)
        out_v[pl.ds(g * L, L)] = accv
        return carry

    lax.fori_loop(0, BPW // L, group, 0)
    pltpu.sync_copy(out_v, out_hbm.at[pl.ds(base, BPW)])


@jax.jit
def kernel(user, item, user_table, item_table):
    mesh = plsc.VectorSubcoreMesh(core_axis_name="c", subcore_axis_name="s")
    run = pl.kernel(
        _body,
        out_type=jax.ShapeDtypeStruct((B,), jnp.float32),
        mesh=mesh,
        compiler_params=pltpu.CompilerParams(
            needs_layout_passes=False,
        ),
        scratch_types=(
            [pltpu.VMEM((BPW,), jnp.int32)] * 2
            + [pltpu.VMEM((D, 128), jnp.float32)] * (2 * NSLOT)
            + [pltpu.VMEM((BPW,), jnp.float32)]
            + [pltpu.SemaphoreType.DMA] * (2 * NSLOT)
        ),
    )
    return run(user.astype(jnp.int32), item.astype(jnp.int32),
               user_table.T, item_table.T)


# speedup vs baseline: 24.4641x; 1.0022x over previous
import functools

import jax
import jax.numpy as jnp
from jax import lax
from jax.experimental import pallas as pl
from jax.experimental.pallas import tpu as pltpu
from jax.experimental.pallas import tpu_sc as plsc

B = 16384
D = 64
NC = 2
NS = 16
NW = NC * NS
BPW = B // NW
L = 16
NSLOT = 4
AHEAD = 3


def _body(user_hbm, item_hbm, utab_hbm, itab_hbm, out_hbm,
          uidx_v, iidx_v, ub0, ub1, ub2, ub3, ib0, ib1, ib2, ib3, out_v,
          su0, su1, su2, su3, si0, si1, si2, si3):
    wid = lax.axis_index("s") * NC + lax.axis_index("c")
    base = wid * BPW

    pltpu.sync_copy(user_hbm.at[pl.ds(base, BPW)], uidx_v)
    pltpu.sync_copy(item_hbm.at[pl.ds(base, BPW)], iidx_v)

    ubufs = (ub0, ub1, ub2, ub3)
    ibufs = (ib0, ib1, ib2, ib3)
    usems = (su0, su1, su2, su3)
    isems = (si0, si1, si2, si3)
    lanes = lax.iota(jnp.int32, L)

    def issue(uj, ij, s):
        ustart = pl.multiple_of((uj >> 7) * 128, 128)
        istart = pl.multiple_of((ij >> 7) * 128, 128)
        for h in range(2):
            rows = pl.ds(h * (D // 2), D // 2)
            pltpu.async_copy(utab_hbm.at[rows, pl.ds(ustart, 128)],
                             ubufs[s].at[rows], usems[s])
            pltpu.async_copy(itab_hbm.at[rows, pl.ds(istart, 128)],
                             ibufs[s].at[rows], isems[s])

    def wait(s):
        pltpu.make_async_copy(utab_hbm.at[:, pl.ds(0, 128)],
                              ubufs[s], usems[s]).wait()
        pltpu.make_async_copy(itab_hbm.at[:, pl.ds(0, 128)],
                              ibufs[s], isems[s]).wait()

    uvec0 = uidx_v[pl.ds(0, L)]
    ivec0 = iidx_v[pl.ds(0, L)]
    for j in range(AHEAD):
        issue(uvec0[j], ivec0[j], j % NSLOT)

    def group(g, carry):
        uvec = uidx_v[pl.ds(g * L, L)]
        ivec = iidx_v[pl.ds(g * L, L)]
        gn = jnp.minimum(g + 1, BPW // L - 1)
        uvecn = uidx_v[pl.ds(gn * L, L)]
        ivecn = iidx_v[pl.ds(gn * L, L)]
        accv = jnp.zeros((L,), jnp.float32)
        for j in range(L):
            s = j % NSLOT
            wait(s)
            ja = j + AHEAD
            if ja < L:
                una, ina = uvec[ja], ivec[ja]
            else:
                una, ina = uvecn[ja - L], ivecn[ja - L]

            @pl.when(g * L + ja < BPW)
            def _():
                issue(una, ina, ja % NSLOT)

            ucol = jnp.full((L,), uvec[j] & 127, jnp.int32)
            icol = jnp.full((L,), ivec[j] & 127, jnp.int32)
            prod = jnp.zeros((L,), jnp.float32)
            for k in range(D // L):
                rows = k * L + lanes
                gu = plsc.load_gather(ubufs[s], [rows, ucol])
                gi = plsc.load_gather(ibufs[s], [rows, icol])
                prod = prod + gu * gi
            dot = jnp.sum(prod)
            accv = jnp.where(lanes == j, dot, accv)
        out_v[pl.ds(g * L, L)] = accv
        return carry

    lax.fori_loop(0, BPW // L, group, 0)
    pltpu.sync_copy(out_v, out_hbm.at[pl.ds(base, BPW)])


@jax.jit
def kernel(user, item, user_table, item_table):
    mesh = plsc.VectorSubcoreMesh(core_axis_name="c", subcore_axis_name="s")
    run = pl.kernel(
        _body,
        out_type=jax.ShapeDtypeStruct((B,), jnp.float32),
        mesh=mesh,
        compiler_params=pltpu.CompilerParams(
            needs_layout_passes=False,
        ),
        scratch_types=(
            [pltpu.VMEM((BPW,), jnp.int32)] * 2
            + [pltpu.VMEM((D, 128), jnp.float32)] * (2 * NSLOT)
            + [pltpu.VMEM((BPW,), jnp.float32)]
            + [pltpu.SemaphoreType.DMA] * (2 * NSLOT)
        ),
    )
    return run(user.astype(jnp.int32), item.astype(jnp.int32),
               user_table.T, item_table.T)
